# baseline (speedup 1.0000x reference)
import functools

import jax
import jax.numpy as jnp
import numpy as np
from jax import lax
from jax.experimental import pallas as pl
from jax.experimental.pallas import tpu as pltpu
from jax.experimental.pallas import tpu_sc as plsc

LAT_R = 256
LAT_C = 256
HALF = LAT_C // 2
TILE_R = 8
TILE_C = 128


@functools.lru_cache(maxsize=None)
def _make_sc_even(num_rows: int):
    info = plsc.get_sparse_core_info()
    NC, NS, L = info.num_cores, info.num_subcores, info.num_lanes
    NW = NC * NS
    rows_per_w = num_rows // NW
    R = 32
    NBUF = 4
    n_chunks = rows_per_w // R
    n_vregs = HALF // L

    mesh = plsc.VectorSubcoreMesh(core_axis_name="c", subcore_axis_name="s")

    @functools.partial(
        pl.kernel,
        mesh=mesh,
        out_type=jax.ShapeDtypeStruct((num_rows * HALF,), jnp.float32),
        scratch_types=(
            [pltpu.VMEM((R * LAT_C,), jnp.float32)] * NBUF
            + [pltpu.VMEM((R * HALF,), jnp.float32)] * NBUF
            + [pltpu.SemaphoreType.DMA] * (2 * NBUF)
        ),
        compiler_params=pltpu.CompilerParams(needs_layout_passes=False),
    )
    def sc_even(x_hbm, even_hbm, *bufs):
        in_v = bufs[:NBUF]
        ev_v = bufs[NBUF : 2 * NBUF]
        in_s = bufs[2 * NBUF : 3 * NBUF]
        ev_s = bufs[3 * NBUF : 4 * NBUF]

        wid = lax.axis_index("s") * NC + lax.axis_index("c")
        base_row = wid * rows_per_w

        two_iota = lax.iota(jnp.int32, L) * 2

        def colmap(c):
            return c + (TILE_R - 1) * TILE_C * (c >> 7)

        ce_c = [colmap(two_iota + 2 * L * kk) for kk in range(n_vregs)]

        def in_slice(c):
            return x_hbm.at[pl.ds((base_row + c * R) * LAT_C, R * LAT_C)]

        def out_slice(c):
            return even_hbm.at[pl.ds((base_row + c * R) * HALF, R * HALF)]

        def compute_chunk(in_ref, ev_ref):
            @plsc.parallel_loop(0, R // 2, 1, unroll=2)
            def _(rp):
                base_e = (rp >> 2) * 2048 + (rp & 3) * 256
                out_e = rp * (2 * HALF)
                out_o = out_e + HALF
                for kk in range(n_vregs):
                    idx_e = ce_c[kk] + base_e
                    ve = plsc.load_gather(in_ref, [idx_e])
                    ev_ref[pl.ds(out_e + kk * L, L)] = ve
                    ve2 = plsc.load_gather(in_ref, [idx_e + (TILE_C + 1)])
                    ev_ref[pl.ds(out_o + kk * L, L)] = ve2

        def step(c, b):
            pltpu.make_async_copy(in_slice(c), in_v[b], in_s[b]).wait()

            @pl.when(c >= NBUF)
            def _():
                pltpu.make_async_copy(ev_v[b], out_slice(c), ev_s[b]).wait()

            compute_chunk(in_v[b], ev_v[b])

            @pl.when(c + NBUF < n_chunks)
            def _():
                pltpu.async_copy(in_slice(c + NBUF), in_v[b], in_s[b])

            pltpu.async_copy(ev_v[b], out_slice(c), ev_s[b])

        for b in range(NBUF):
            pltpu.async_copy(in_slice(b), in_v[b], in_s[b])

        def g_body(g, carry):
            for b in range(NBUF):
                step(g * NBUF + b, b)
            return carry

        lax.fori_loop(0, n_chunks // NBUF, g_body, 0)

        for b in range(NBUF):
            pltpu.make_async_copy(ev_v[b], out_slice(0), ev_s[b]).wait()

    return sc_even


def _odd_matrices():
    p1 = np.zeros((LAT_C, HALF), np.float32)
    p2 = np.zeros((LAT_C, HALF), np.float32)
    for j in range(HALF):
        p1[2 * j + 1, j] = 1.0
        p2[(2 * j + 2) % LAT_C, j] = 1.0
    return jnp.asarray(p1), jnp.asarray(p2)


def _tc_odd_body(x_ref, p1_ref, p2_ref, o_ref):
    b = x_ref.shape[0]
    xb = x_ref[...].reshape(b * LAT_R, LAT_C)
    dot = functools.partial(
        jax.lax.dot_general,
        dimension_numbers=(((1,), (0,)), ((), ())),
        precision=jax.lax.Precision.HIGHEST,
        preferred_element_type=jnp.float32,
    )
    o1 = dot(xb, p1_ref[...])
    o2 = dot(xb, p2_ref[...])
    par = lax.broadcasted_iota(jnp.int32, o1.shape, 0) & 1
    o_ref[...] = jnp.where(par == 1, o2, o1).reshape(b, LAT_R, HALF)


def _tc_odd(x):
    num_samples = x.shape[0]
    B = 8
    p1, p2 = _odd_matrices()
    return pl.pallas_call(
        _tc_odd_body,
        grid=(num_samples // B,),
        in_specs=[
            pl.BlockSpec((B, LAT_R, LAT_C), lambda i: (i, 0, 0)),
            pl.BlockSpec((LAT_C, HALF), lambda i: (0, 0)),
            pl.BlockSpec((LAT_C, HALF), lambda i: (0, 0)),
        ],
        out_specs=pl.BlockSpec((B, LAT_R, HALF), lambda i: (i, 0, 0)),
        out_shape=jax.ShapeDtypeStruct((num_samples, LAT_R, HALF), jnp.float32),
        compiler_params=pltpu.CompilerParams(dimension_semantics=("arbitrary",)),
    )(x, p1, p2)


def kernel(x):
    num_samples = x.shape[0]
    num_rows = num_samples * LAT_R
    x_view = (
        x.reshape(num_rows // TILE_R, TILE_R, 2, TILE_C)
        .transpose(0, 2, 1, 3)
        .reshape(num_rows * LAT_C)
    )
    even = _make_sc_even(num_rows)(x_view)
    odd = _tc_odd(x)
    return (even.reshape(num_samples, LAT_R, HALF), odd)


# speedup vs baseline: 2.2345x; 2.2345x over previous
import functools

import jax
import jax.numpy as jnp
from jax import lax
from jax.experimental import pallas as pl
from jax.experimental.pallas import tpu as pltpu
from jax.experimental.pallas import tpu_sc as plsc

LAT_R = 256
LAT_C = 256
HALF = LAT_C // 2
TILE_R = 8
TILE_C = 128


@functools.lru_cache(maxsize=None)
def _make_sc_split(num_rows: int):
    info = plsc.get_sparse_core_info()
    NC, NS, L = info.num_cores, info.num_subcores, info.num_lanes
    NW = NC * NS
    rows_per_w = num_rows // NW
    R = 32
    NBUF = 4
    n_chunks = rows_per_w // R
    n_vregs = HALF // L

    mesh = plsc.VectorSubcoreMesh(core_axis_name="c", subcore_axis_name="s")

    @functools.partial(
        pl.kernel,
        mesh=mesh,
        out_type=(
            jax.ShapeDtypeStruct((num_rows * HALF,), jnp.float32),
            jax.ShapeDtypeStruct((num_rows * HALF,), jnp.float32),
        ),
        scratch_types=(
            [pltpu.VMEM((R * LAT_C,), jnp.float32)] * NBUF
            + [pltpu.VMEM((R * HALF,), jnp.float32)] * (2 * NBUF)
            + [pltpu.SemaphoreType.DMA] * (3 * NBUF)
        ),
        compiler_params=pltpu.CompilerParams(needs_layout_passes=False),
    )
    def sc_split(x_hbm, even_hbm, odd_hbm, *bufs):
        in_v = bufs[:NBUF]
        ev_v = bufs[NBUF : 2 * NBUF]
        od_v = bufs[2 * NBUF : 3 * NBUF]
        in_s = bufs[3 * NBUF : 4 * NBUF]
        ev_s = bufs[4 * NBUF : 5 * NBUF]
        od_s = bufs[5 * NBUF : 6 * NBUF]

        wid = lax.axis_index("s") * NC + lax.axis_index("c")
        base_row = wid * rows_per_w

        two_iota = lax.iota(jnp.int32, L) * 2

        def colmap(c):
            return c + (TILE_R - 1) * TILE_C * (c >> 7)

        ce_c = [colmap(two_iota + 2 * L * kk) for kk in range(n_vregs)]
        cr3_c = colmap(two_iota + 2 * L * 3 + 2)
        cr7_c = colmap(jnp.bitwise_and(two_iota + 2 * L * 7 + 2, LAT_C - 1))

        def in_slice(c):
            return x_hbm.at[pl.ds((base_row + c * R) * LAT_C, R * LAT_C)]

        def out_slice(hbm, c):
            return hbm.at[pl.ds((base_row + c * R) * HALF, R * HALF)]

        def compute_chunk(in_ref, ev_ref, od_ref):
            @plsc.parallel_loop(0, R // 2, 1, unroll=1)
            def _(rp):
                base_e = (rp >> 2) * 2048 + (rp & 3) * 256
                base_o = base_e + TILE_C
                out_e = rp * (2 * HALF)
                out_o = out_e + HALF
                for kk in range(n_vregs):
                    idx_e = ce_c[kk] + base_e
                    if kk == 3:
                        idx_r = cr3_c + base_o
                    elif kk == 7:
                        idx_r = cr7_c + base_o
                    else:
                        idx_r = idx_e + (TILE_C + 2)
                    ve = plsc.load_gather(in_ref, [idx_e])
                    vo = plsc.load_gather(in_ref, [idx_e + 1])
                    ev_ref[pl.ds(out_e + kk * L, L)] = ve
                    od_ref[pl.ds(out_e + kk * L, L)] = vo
                    ve2 = plsc.load_gather(in_ref, [idx_e + (TILE_C + 1)])
                    vo2 = plsc.load_gather(in_ref, [idx_r])
                    ev_ref[pl.ds(out_o + kk * L, L)] = ve2
                    od_ref[pl.ds(out_o + kk * L, L)] = vo2

        def step(c, b):
            pltpu.make_async_copy(in_slice(c), in_v[b], in_s[b]).wait()

            @pl.when(c >= NBUF)
            def _():
                pltpu.make_async_copy(ev_v[b], out_slice(even_hbm, c), ev_s[b]).wait()
                pltpu.make_async_copy(od_v[b], out_slice(odd_hbm, c), od_s[b]).wait()

            compute_chunk(in_v[b], ev_v[b], od_v[b])

            @pl.when(c + NBUF < n_chunks)
            def _():
                pltpu.async_copy(in_slice(c + NBUF), in_v[b], in_s[b])

            pltpu.async_copy(ev_v[b], out_slice(even_hbm, c), ev_s[b])
            pltpu.async_copy(od_v[b], out_slice(odd_hbm, c), od_s[b])

        for b in range(NBUF):
            pltpu.async_copy(in_slice(b), in_v[b], in_s[b])

        def g_body(g, carry):
            for b in range(NBUF):
                step(g * NBUF + b, b)
            return carry

        lax.fori_loop(0, n_chunks // NBUF, g_body, 0)

        for b in range(NBUF):
            pltpu.make_async_copy(ev_v[b], out_slice(even_hbm, 0), ev_s[b]).wait()
            pltpu.make_async_copy(od_v[b], out_slice(odd_hbm, 0), od_s[b]).wait()

    return sc_split


def kernel(x):
    num_samples = x.shape[0]
    num_rows = num_samples * LAT_R
    x_view = (
        x.reshape(num_rows // TILE_R, TILE_R, 2, TILE_C)
        .transpose(0, 2, 1, 3)
        .reshape(num_rows * LAT_C)
    )
    even, odd = _make_sc_split(num_rows)(x_view)
    shape = (num_samples, LAT_R, HALF)
    return (even.reshape(shape), odd.reshape(shape))
